# baseline (speedup 1.0000x reference)
import functools

import jax
import jax.numpy as jnp
from jax import lax
from jax.experimental import pallas as pl
from jax.experimental.pallas import tpu as pltpu
from jax.experimental.pallas import tpu_sc as plsc

_EMBED = 32
_BATCH = 16384
_FIELDS = 26
_VOCAB = 1000000
_B = _BATCH * _FIELDS
_NC = 2
_NS = 16
_NW = _NC * _NS
_BPW = _B // _NW
_CHUNK = 1664
_NCHUNK = _BPW // _CHUNK

_mesh = plsc.VectorSubcoreMesh(core_axis_name="c", subcore_axis_name="s")


@functools.partial(
    pl.kernel,
    mesh=_mesh,
    out_type=jax.ShapeDtypeStruct((_B, _EMBED), jnp.float32),
    scratch_types=[
        pltpu.VMEM((_NCHUNK, _CHUNK), jnp.int32),
        pltpu.VMEM((2, _CHUNK, _EMBED), jnp.float32),
        pltpu.SemaphoreType.DMA,
        pltpu.SemaphoreType.DMA,
    ],
    compiler_params=pltpu.CompilerParams(use_tc_tiling_on_sc=False),
)
def _gather(idx_hbm, table_hbm, out_hbm, idx_v, rows_v, sem0, sem1):
    wid = lax.axis_index("s") * _NC + lax.axis_index("c")
    base = wid * _BPW
    sems = (sem0, sem1)
    pltpu.sync_copy(idx_hbm.at[pl.ds(wid * _NCHUNK, _NCHUNK)], idx_v)
    pltpu.async_copy(table_hbm.at[idx_v.at[0]], rows_v.at[0], sems[0])
    for i in range(_NCHUNK):
        if i + 1 < _NCHUNK:
            pltpu.async_copy(
                table_hbm.at[idx_v.at[i + 1]], rows_v.at[(i + 1) % 2],
                sems[(i + 1) % 2])
        pltpu.make_async_copy(
            table_hbm.at[idx_v.at[i]], rows_v.at[i % 2], sems[i % 2]).wait()
        pltpu.sync_copy(rows_v.at[i % 2],
                        out_hbm.at[pl.ds(base + i * _CHUNK, _CHUNK)])


_TLANES = 8192
_TGRID = -(-_VOCAB // _TLANES)


def _tc_table_body(x_ref, o_ref):
    x = x_ref[...]
    o_ref[...] = (x.reshape(_EMBED, _TLANES // 4, 4)
                  .transpose(1, 2, 0)
                  .reshape(_TLANES // 4, 128))


_tc_table = pl.pallas_call(
    _tc_table_body,
    grid=(_TGRID,),
    in_specs=[pl.BlockSpec((_EMBED, _TLANES), lambda c: (0, c))],
    out_specs=pl.BlockSpec((_TLANES // 4, 128), lambda c: (c, 0)),
    out_shape=jax.ShapeDtypeStruct((_VOCAB * _EMBED // 128, 128),
                                   jnp.float32),
)


def _tc_format_body(x_ref, o_ref):
    x = x_ref[...]
    o_ref[...] = (x.reshape(512, 4, _EMBED)
                  .transpose(2, 0, 1)
                  .reshape(1, _EMBED, 2048))


_tc_format = pl.pallas_call(
    _tc_format_body,
    grid=(_FIELDS, _BATCH // 2048),
    in_specs=[pl.BlockSpec((512, 128), lambda f, c: (f * 8 + c, 0))],
    out_specs=pl.BlockSpec((1, _EMBED, 2048), lambda f, c: (f, 0, c)),
    out_shape=jax.ShapeDtypeStruct((_FIELDS, _EMBED, _BATCH), jnp.float32),
)


def kernel(x, table):
    idx = x.T.reshape(_B // _CHUNK, _CHUNK).astype(jnp.int32)
    table_lin = _tc_table(table.T).reshape(_VOCAB, _EMBED)
    lin = _gather(idx, table_lin)
    out_t = _tc_format(lin.reshape(_B * _EMBED // 128, 128))
    return jnp.transpose(out_t, (2, 0, 1))


# speedup vs baseline: 4.6760x; 4.6760x over previous
import functools

import jax
import jax.numpy as jnp
from jax import lax
from jax.experimental import pallas as pl
from jax.experimental.pallas import tpu as pltpu
from jax.experimental.pallas import tpu_sc as plsc

_EMBED = 32
_B = 16384 * 26
_NC = 2
_NS = 16
_NW = _NC * _NS
_BPW = _B // _NW
_CHUNK = 1664
_NCHUNK = _BPW // _CHUNK

_mesh = plsc.VectorSubcoreMesh(core_axis_name="c", subcore_axis_name="s")


@functools.partial(
    pl.kernel,
    mesh=_mesh,
    out_type=jax.ShapeDtypeStruct((_B, _EMBED), jnp.float32),
    scratch_types=[
        pltpu.VMEM((_NCHUNK, _CHUNK), jnp.int32),
        pltpu.VMEM((2, _CHUNK, _EMBED), jnp.float32),
        pltpu.SemaphoreType.DMA,
        pltpu.SemaphoreType.DMA,
    ],
    compiler_params=pltpu.CompilerParams(use_tc_tiling_on_sc=False),
)
def _gather(idx_hbm, table_hbm, out_hbm, idx_v, rows_v, sem0, sem1):
    wid = lax.axis_index("s") * _NC + lax.axis_index("c")
    base = wid * _BPW
    sems = (sem0, sem1)
    pltpu.sync_copy(idx_hbm.at[pl.ds(wid * _NCHUNK, _NCHUNK)], idx_v)
    pltpu.async_copy(table_hbm.at[idx_v.at[0]], rows_v.at[0], sems[0])
    for i in range(_NCHUNK):
        if i + 1 < _NCHUNK:
            pltpu.async_copy(
                table_hbm.at[idx_v.at[i + 1]], rows_v.at[(i + 1) % 2],
                sems[(i + 1) % 2])
        pltpu.make_async_copy(
            table_hbm.at[idx_v.at[i]], rows_v.at[i % 2], sems[i % 2]).wait()
        pltpu.sync_copy(rows_v.at[i % 2],
                        out_hbm.at[pl.ds(base + i * _CHUNK, _CHUNK)])


def kernel(x, table):
    idx = x.T.reshape(_B // _CHUNK, _CHUNK).astype(jnp.int32)
    out = _gather(idx, table)
    return jnp.transpose(out.reshape(26, 16384, _EMBED), (1, 0, 2))
